# baseline (speedup 1.0000x reference)
import jax
import jax.numpy as jnp
from jax import lax
from jax.experimental import pallas as pl
from jax.experimental.pallas import tpu as pltpu
from jax.experimental.pallas import tpu_sc as plsc

NUM_BINS = 100
L_OUT = NUM_BINS + 2
EPS = 1e-3

NC, NS, LANES = 2, 16, 16
NW = NC * NS

PB = 112
ACCW = PB * LANES
SEG = 128
PART = 3 * SEG

CH = 16384
LUT = 8192


def _phase1(probs, targs):
    n = probs.shape[0]
    assert n % (NW * CH) == 0, n
    epw = n // NW
    nch = epw // CH

    def body(p_hbm, t_hbm, part_hbm, pbuf0, pbuf1, tbuf0, tbuf1,
             acc_t, acc_p, acc_c, part_v, lut_e, sp0, sp1, st0, st1):
        cid = lax.axis_index("c")
        sid = lax.axis_index("s")
        wid = sid * NC + cid
        base = wid * epw

        zeros16 = jnp.zeros((LANES,), jnp.float32)
        ones16 = jnp.ones((LANES,), jnp.float32)
        ones16i = jnp.ones((LANES,), jnp.int32)
        zeros16i = jnp.zeros((LANES,), jnp.int32)
        lane = lax.iota(jnp.int32, LANES)
        step_f = jnp.float32(0.01)
        hundred = jnp.float32(100.0)
        one_f = jnp.float32(1.0)
        half = jnp.float32(0.5)
        flut = jnp.float32(LUT)
        sixteen_i = jnp.full((LANES,), LANES, jnp.int32)

        sems_p = (sp0, sp1)
        sems_t = (st0, st1)
        pbufs = (pbuf0, pbuf1)
        tbufs = (tbuf0, tbuf1)

        def start(g, b):
            off = base + g * CH
            pltpu.async_copy(p_hbm.at[pl.ds(off, CH)], pbufs[b], sems_p[b])
            pltpu.async_copy(t_hbm.at[pl.ds(off, CH)], tbufs[b], sems_t[b])

        def wait(g, b):
            off = base + g * CH
            pltpu.make_async_copy(
                p_hbm.at[pl.ds(off, CH)], pbufs[b], sems_p[b]).wait()
            pltpu.make_async_copy(
                t_hbm.at[pl.ds(off, CH)], tbufs[b], sems_t[b]).wait()

        start(0, 0)
        start(1, 1)

        @pl.loop(0, ACCW // LANES)
        def _zero(i):
            o = i * LANES
            acc_t[pl.ds(o, LANES)] = zeros16
            acc_p[pl.ds(o, LANES)] = zeros16
            acc_c[pl.ds(o, LANES)] = zeros16

        inv_lut = jnp.float32(1.0 / LUT)

        @pl.loop(0, LUT // LANES)
        def _lut(v):
            kv = v * LANES + lane
            pmin = kv.astype(jnp.float32) * inv_lut
            ji = (pmin * hundred).astype(jnp.int32)
            jf = ji.astype(jnp.float32)
            b0 = jnp.where(jf * step_f < pmin, ones16i, zeros16i)
            b1 = jnp.where((jf + one_f) * step_f < pmin, ones16i, zeros16i)
            j0 = ji + b0 + b1
            lut_e[pl.ds(v * LANES, LANES)] = j0.astype(jnp.float32) * step_f

        @pl.loop(0, nch, step=2)
        def _chunk(g):
            for b in range(2):
                gg = g + b
                wait(gg, b)
                pb = pbufs[b]
                tb = tbufs[b]

                @plsc.parallel_loop(0, CH // LANES, unroll=8)
                def _vec(v):
                    o = v * LANES
                    pv = pb[pl.ds(o, LANES)]
                    tv = tb[pl.ds(o, LANES)]
                    ki = jnp.minimum((pv * flut).astype(jnp.int32), LUT - 1)
                    ev = plsc.load_gather(lut_e, [ki])
                    ji = (ev * hundred + half).astype(jnp.int32)
                    corr = jnp.where(ev < pv, sixteen_i, zeros16i)
                    fidx = ji * LANES + corr + lane
                    plsc.addupdate_scatter(acc_c, [fidx], ones16)
                    plsc.addupdate_scatter(acc_p, [fidx], pv)
                    plsc.addupdate_scatter(acc_t, [fidx], tv)

                nxt = gg + 2

                @pl.when(nxt < nch)
                def _():
                    start(nxt, b)

        @pl.loop(0, PART // LANES)
        def _zpart(i):
            part_v[pl.ds(i * LANES, LANES)] = zeros16

        for q, aref in enumerate((acc_t, acc_p, acc_c)):
            @pl.loop(0, PB // LANES)
            def _red(gr, _q=q, _aref=aref):
                bb = gr * (LANES * LANES)
                s = zeros16
                for r in range(LANES):
                    s = s + plsc.load_gather(_aref, [lane * LANES + (bb + r)])
                part_v[pl.ds(_q * SEG + gr * LANES, LANES)] = s

        pltpu.sync_copy(part_v, part_hbm.at[pl.ds(wid * PART, PART)])

    mesh = plsc.VectorSubcoreMesh(core_axis_name="c", subcore_axis_name="s",
                                  num_cores=NC, num_subcores=NS)
    f = pl.kernel(
        body,
        out_type=jax.ShapeDtypeStruct((NW * PART,), jnp.float32),
        mesh=mesh,
        compiler_params=pltpu.CompilerParams(needs_layout_passes=False),
        scratch_types=[
            pltpu.VMEM((CH,), jnp.float32),
            pltpu.VMEM((CH,), jnp.float32),
            pltpu.VMEM((CH,), jnp.float32),
            pltpu.VMEM((CH,), jnp.float32),
            pltpu.VMEM((ACCW,), jnp.float32),
            pltpu.VMEM((ACCW,), jnp.float32),
            pltpu.VMEM((ACCW,), jnp.float32),
            pltpu.VMEM((PART,), jnp.float32),
            pltpu.VMEM((LUT,), jnp.float32),
            pltpu.SemaphoreType.DMA,
            pltpu.SemaphoreType.DMA,
            pltpu.SemaphoreType.DMA,
            pltpu.SemaphoreType.DMA,
        ],
    )
    return f(probs, targs)


def _phase2(part):
    def body(part_hbm, out_hbm, buf, sbuf, obuf):
        cid = lax.axis_index("c")
        sid = lax.axis_index("s")
        wid = sid * NC + cid

        @pl.when(wid == 0)
        def _():
            pltpu.sync_copy(part_hbm, buf)
            zeros16 = jnp.zeros((LANES,), jnp.float32)

            @pl.loop(0, PART // LANES)
            def _sum(c):
                s = zeros16
                for w in range(NW):
                    s = s + buf[pl.ds(w * PART + c * LANES, LANES)]
                sbuf[pl.ds(c * LANES, LANES)] = s

            @pl.loop(0, SEG // LANES)
            def _fin(k):
                o = k * LANES
                cnt = sbuf[pl.ds(2 * SEG + o, LANES)] + jnp.float32(EPS)
                obuf[pl.ds(o, LANES)] = sbuf[pl.ds(o, LANES)] / cnt
                obuf[pl.ds(SEG + o, LANES)] = (
                    sbuf[pl.ds(SEG + o, LANES)] / cnt)
                obuf[pl.ds(2 * SEG + o, LANES)] = cnt

            pltpu.sync_copy(obuf, out_hbm)

    mesh = plsc.VectorSubcoreMesh(core_axis_name="c", subcore_axis_name="s",
                                  num_cores=NC, num_subcores=NS)
    f = pl.kernel(
        body,
        out_type=jax.ShapeDtypeStruct((PART,), jnp.float32),
        mesh=mesh,
        compiler_params=pltpu.CompilerParams(needs_layout_passes=False),
        scratch_types=[
            pltpu.VMEM((NW * PART,), jnp.float32),
            pltpu.VMEM((PART,), jnp.float32),
            pltpu.VMEM((PART,), jnp.float32),
        ],
    )
    return f(part)


def kernel(probabilities, predictions, targets):
    del predictions
    part = _phase1(probabilities, targets)
    out = _phase2(part)
    return jnp.stack([out[0:L_OUT], out[SEG:SEG + L_OUT],
                      out[2 * SEG:2 * SEG + L_OUT]])


# speedup vs baseline: 1.2736x; 1.2736x over previous
import jax
import jax.numpy as jnp
from jax import lax
from jax.experimental import pallas as pl
from jax.experimental.pallas import tpu as pltpu
from jax.experimental.pallas import tpu_sc as plsc

NUM_BINS = 100
L_OUT = NUM_BINS + 2
EPS = 1e-3

NC, NS, LANES = 2, 16, 16
NW = NC * NS

PB = 112
ACCW = PB * LANES
SEG = 128
PART = 3 * SEG

CH = 16384
LUT = 8192


def _phase1(probs, targs):
    n = probs.shape[0]
    assert n % (NW * CH) == 0, n
    epw = n // NW
    nch = epw // CH

    def body(p_hbm, t_hbm, part_hbm, pbuf0, pbuf1, tbuf0, tbuf1,
             acc_t, acc_p, acc_c, acc_t1, acc_p1, acc_c1,
             part_v, sp0, sp1, st0, st1):
        cid = lax.axis_index("c")
        sid = lax.axis_index("s")
        wid = sid * NC + cid
        base = wid * epw

        zeros16 = jnp.zeros((LANES,), jnp.float32)
        ones16 = jnp.ones((LANES,), jnp.float32)
        ones16i = jnp.ones((LANES,), jnp.int32)
        zeros16i = jnp.zeros((LANES,), jnp.int32)
        lane = lax.iota(jnp.int32, LANES)
        step_f = jnp.float32(0.01)
        hundred = jnp.float32(100.0)
        one_f = jnp.float32(1.0)
        half = jnp.float32(0.5)
        flut = jnp.float32(LUT)
        sixteen_i = jnp.full((LANES,), LANES, jnp.int32)

        sems_p = (sp0, sp1)
        sems_t = (st0, st1)
        pbufs = (pbuf0, pbuf1)
        tbufs = (tbuf0, tbuf1)

        def start(g, b):
            off = base + g * CH
            pltpu.async_copy(p_hbm.at[pl.ds(off, CH)], pbufs[b], sems_p[b])
            pltpu.async_copy(t_hbm.at[pl.ds(off, CH)], tbufs[b], sems_t[b])

        def wait(g, b):
            off = base + g * CH
            pltpu.make_async_copy(
                p_hbm.at[pl.ds(off, CH)], pbufs[b], sems_p[b]).wait()
            pltpu.make_async_copy(
                t_hbm.at[pl.ds(off, CH)], tbufs[b], sems_t[b]).wait()

        start(0, 0)
        start(1, 1)

        @pl.loop(0, ACCW // LANES)
        def _zero(i):
            o = i * LANES
            acc_t[pl.ds(o, LANES)] = zeros16
            acc_p[pl.ds(o, LANES)] = zeros16
            acc_c[pl.ds(o, LANES)] = zeros16
            acc_t1[pl.ds(o, LANES)] = zeros16
            acc_p1[pl.ds(o, LANES)] = zeros16
            acc_c1[pl.ds(o, LANES)] = zeros16

        banks = ((acc_c, acc_p, acc_t), (acc_c1, acc_p1, acc_t1))

        @pl.loop(0, nch, step=2)
        def _chunk(g):
            for b in range(2):
                gg = g + b
                wait(gg, b)
                pb = pbufs[b]
                tb = tbufs[b]

                @plsc.parallel_loop(0, CH // LANES, step=2, unroll=4)
                def _vec(v):
                    for sub in range(2):
                        a_c, a_p, a_t = banks[sub]
                        o = (v + sub) * LANES
                        pv = pb[pl.ds(o, LANES)]
                        tv = tb[pl.ds(o, LANES)]
                        ji = (pv * hundred).astype(jnp.int32)
                        jf = ji.astype(jnp.float32)
                        b0 = jnp.where(jf * step_f < pv, ones16i, zeros16i)
                        b1 = jnp.where((jf + one_f) * step_f < pv, ones16i,
                                       zeros16i)
                        idx = ji + b0 + b1
                        fidx = idx * LANES + lane
                        plsc.addupdate_scatter(a_c, [fidx], ones16)
                        plsc.addupdate_scatter(a_p, [fidx], pv)
                        plsc.addupdate_scatter(a_t, [fidx], tv)

                nxt = gg + 2

                @pl.when(nxt < nch)
                def _():
                    start(nxt, b)

        @pl.loop(0, ACCW // LANES)
        def _merge(i):
            o = i * LANES
            acc_t[pl.ds(o, LANES)] = acc_t[pl.ds(o, LANES)] + acc_t1[pl.ds(o, LANES)]
            acc_p[pl.ds(o, LANES)] = acc_p[pl.ds(o, LANES)] + acc_p1[pl.ds(o, LANES)]
            acc_c[pl.ds(o, LANES)] = acc_c[pl.ds(o, LANES)] + acc_c1[pl.ds(o, LANES)]

        @pl.loop(0, PART // LANES)
        def _zpart(i):
            part_v[pl.ds(i * LANES, LANES)] = zeros16

        for q, aref in enumerate((acc_t, acc_p, acc_c)):
            @pl.loop(0, PB // LANES)
            def _red(gr, _q=q, _aref=aref):
                bb = gr * (LANES * LANES)
                s = zeros16
                for r in range(LANES):
                    s = s + plsc.load_gather(_aref, [lane * LANES + (bb + r)])
                part_v[pl.ds(_q * SEG + gr * LANES, LANES)] = s

        pltpu.sync_copy(part_v, part_hbm.at[pl.ds(wid * PART, PART)])

    mesh = plsc.VectorSubcoreMesh(core_axis_name="c", subcore_axis_name="s",
                                  num_cores=NC, num_subcores=NS)
    f = pl.kernel(
        body,
        out_type=jax.ShapeDtypeStruct((NW * PART,), jnp.float32),
        mesh=mesh,
        compiler_params=pltpu.CompilerParams(needs_layout_passes=False),
        scratch_types=[
            pltpu.VMEM((CH,), jnp.float32),
            pltpu.VMEM((CH,), jnp.float32),
            pltpu.VMEM((CH,), jnp.float32),
            pltpu.VMEM((CH,), jnp.float32),
            pltpu.VMEM((ACCW,), jnp.float32),
            pltpu.VMEM((ACCW,), jnp.float32),
            pltpu.VMEM((ACCW,), jnp.float32),
            pltpu.VMEM((ACCW,), jnp.float32),
            pltpu.VMEM((ACCW,), jnp.float32),
            pltpu.VMEM((ACCW,), jnp.float32),
            pltpu.VMEM((PART,), jnp.float32),
            pltpu.SemaphoreType.DMA,
            pltpu.SemaphoreType.DMA,
            pltpu.SemaphoreType.DMA,
            pltpu.SemaphoreType.DMA,
        ],
    )
    return f(probs, targs)


def _phase2(part):
    def body(part_hbm, out_hbm, buf, sbuf, obuf):
        cid = lax.axis_index("c")
        sid = lax.axis_index("s")
        wid = sid * NC + cid

        @pl.when(wid == 0)
        def _():
            pltpu.sync_copy(part_hbm, buf)
            zeros16 = jnp.zeros((LANES,), jnp.float32)

            @pl.loop(0, PART // LANES)
            def _sum(c):
                s = zeros16
                for w in range(NW):
                    s = s + buf[pl.ds(w * PART + c * LANES, LANES)]
                sbuf[pl.ds(c * LANES, LANES)] = s

            @pl.loop(0, SEG // LANES)
            def _fin(k):
                o = k * LANES
                cnt = sbuf[pl.ds(2 * SEG + o, LANES)] + jnp.float32(EPS)
                obuf[pl.ds(o, LANES)] = sbuf[pl.ds(o, LANES)] / cnt
                obuf[pl.ds(SEG + o, LANES)] = (
                    sbuf[pl.ds(SEG + o, LANES)] / cnt)
                obuf[pl.ds(2 * SEG + o, LANES)] = cnt

            pltpu.sync_copy(obuf, out_hbm)

    mesh = plsc.VectorSubcoreMesh(core_axis_name="c", subcore_axis_name="s",
                                  num_cores=NC, num_subcores=NS)
    f = pl.kernel(
        body,
        out_type=jax.ShapeDtypeStruct((PART,), jnp.float32),
        mesh=mesh,
        compiler_params=pltpu.CompilerParams(needs_layout_passes=False),
        scratch_types=[
            pltpu.VMEM((NW * PART,), jnp.float32),
            pltpu.VMEM((PART,), jnp.float32),
            pltpu.VMEM((PART,), jnp.float32),
        ],
    )
    return f(part)


def kernel(probabilities, predictions, targets):
    del predictions
    part = _phase1(probabilities, targets)
    out = _phase2(part)
    return jnp.stack([out[0:L_OUT], out[SEG:SEG + L_OUT],
                      out[2 * SEG:2 * SEG + L_OUT]])
